# baseline (speedup 1.0000x reference)
import functools

import jax
import jax.numpy as jnp
from jax import lax
from jax.experimental import pallas as pl
from jax.experimental.pallas import tpu as pltpu
from jax.experimental.pallas import tpu_sc as plsc

N = 10000
E = 320000
D_IN = 128
OC = 21
H1 = 16 * OC
H2 = 8 * OC
N_ACT = 8
MAX_LOGSTD = 10.0

NC, NS = 2, 16
NW = NC * NS
CHUNK = 128
CPT = 80
E_PAD = NW * CPT * CHUNK
ACC_ROWS = 10240
RPT = ACC_ROWS // NS
BN = 1280
ZPADC = 32


def _make_segsum(d, with_hist, chunk, rr, ir, cpt0=None):
    tot = E_PAD // NS // chunk
    U = max(rr, ir)
    if cpt0 is None:
        cpt0 = tot // 2
    cpt1 = tot - cpt0
    assert cpt0 % U == 0 and cpt1 % U == 0 and ir >= 4
    mesh = plsc.VectorSubcoreMesh(core_axis_name="c", subcore_axis_name="s")
    out_type = [jax.ShapeDtypeStruct((NC, ACC_ROWS, d), jnp.float32)]
    if with_hist:
        out_type.append(jax.ShapeDtypeStruct((NW * ACC_ROWS,), jnp.float32))
    scratch = (
        [pltpu.VMEM((chunk,), jnp.int32) for _ in range(ir)]
        + [pltpu.VMEM((chunk,), jnp.int32) for _ in range(ir)]
        + [pltpu.VMEM((chunk, d), jnp.float32) for _ in range(rr)]
        + [pltpu.VMEM_SHARED((ACC_ROWS, d), jnp.float32)]
        + [pltpu.SemaphoreType.DMA] * (2 * ir + 2 * rr)
    )
    if with_hist:
        scratch.append(pltpu.VMEM((ACC_ROWS,), jnp.float32))

    def body(feat, srcf, dstf, zrows, *rest):
        if with_hist:
            (agg_o, deg_o), refs = rest[:2], rest[2:]
        else:
            (agg_o,), refs = rest[:1], rest[1:]
        srci = refs[0:ir]
        dsti = refs[ir:2 * ir]
        rows = refs[2 * ir:2 * ir + rr]
        acc = refs[2 * ir + rr]
        sems = refs[2 * ir + rr + 1:]
        isems = sems[0:ir]
        jsems = sems[ir:2 * ir]
        gsem = sems[2 * ir:2 * ir + rr]
        ssem = sems[2 * ir + rr:2 * ir + 2 * rr]
        if with_hist:
            histv = refs[-1]
        cid = lax.axis_index("c")
        sid = lax.axis_index("s")
        wid = cid * NS + sid
        cpt = jnp.where(cid == 0, cpt0, cpt1)
        cbase = jnp.where(cid == 0, sid * cpt0, NS * cpt0 + sid * cpt1)

        pltpu.sync_copy(zrows, acc.at[pl.ds(sid * RPT, RPT)])
        if with_hist:
            @pl.loop(0, ACC_ROWS // 16)
            def _(i):
                histv[pl.ds(i * 16, 16)] = jnp.zeros((16,), jnp.float32)

        plsc.subcore_barrier()

        ebase = cbase * chunk

        def i_start(kt, b):
            off = ebase + kt * chunk
            pltpu.async_copy(srcf.at[pl.ds(off, chunk)], srci[b], isems[b])
            pltpu.async_copy(dstf.at[pl.ds(off, chunk)], dsti[b], jsems[b])

        def i_wait(kt, b):
            off = ebase + kt * chunk
            pltpu.make_async_copy(srcf.at[pl.ds(off, chunk)], srci[b],
                                  isems[b]).wait()
            pltpu.make_async_copy(dstf.at[pl.ds(off, chunk)], dsti[b],
                                  jsems[b]).wait()

        def g_start(b, rb):
            pltpu.async_copy(feat.at[srci[b]], rows[rb], gsem[rb])

        def g_wait(b, rb):
            pltpu.make_async_copy(feat.at[srci[b]], rows[rb], gsem[rb]).wait()

        def s_start(b, rb):
            pltpu.async_copy(rows[rb], acc.at[dsti[b]], ssem[rb], add=True)

        def s_wait(b, rb):
            pltpu.make_async_copy(rows[rb], acc.at[dsti[b]], ssem[rb]).wait()

        ones = jnp.ones((16,), jnp.float32)

        def hist_upd(b):
            if with_hist:
                for j in range(chunk // 16):
                    idx = dsti[b][pl.ds(j * 16, 16)]
                    plsc.addupdate_scatter(histv, [idx], ones)

        def emit_step(sk, kt):
            if sk >= rr:
                s_wait((sk - rr) % ir, sk % rr)
            i_start(kt + 2, (sk + 2) % ir)
            i_wait(kt, sk % ir)
            g_start(sk % ir, sk % rr)
            hist_upd(sk % ir)
            if sk >= 1:
                g_wait((sk - 1) % ir, (sk - 1) % rr)
                s_start((sk - 1) % ir, (sk - 1) % rr)

        i_start(0, 0)
        i_start(1, 1)
        for k in range(U):
            emit_step(k, k)

        @pl.loop(1, cpt // U)
        def _(g):
            for u in range(U):
                emit_step(U + u, g * U + u)

        g_wait(ir - 1, rr - 1)
        s_start(ir - 1, rr - 1)
        i_wait(cpt, 0)
        i_wait(cpt + 1, 1)
        for j in range(rr):
            s_wait((ir - rr + j) % ir, j)

        if with_hist:
            pltpu.sync_copy(histv,
                            deg_o.at[pl.ds(wid * ACC_ROWS, ACC_ROWS)])

        plsc.subcore_barrier()
        pltpu.sync_copy(acc.at[pl.ds(sid * RPT, RPT)],
                        agg_o.at[cid, pl.ds(sid * RPT, RPT)])

    return pl.kernel(body, out_type=out_type, mesh=mesh,
                     scratch_types=scratch,
                     compiler_params=pltpu.CompilerParams(
                         needs_layout_passes=False,
                         use_tc_tiling_on_sc=False))


def _make_edge_gather(cpt0=None):
    mesh = plsc.VectorSubcoreMesh(core_axis_name="c", subcore_axis_name="s")
    out_type = [jax.ShapeDtypeStruct((E_PAD, ZPADC), jnp.float32),
                jax.ShapeDtypeStruct((E_PAD, ZPADC), jnp.float32)]
    scratch = (
        [pltpu.VMEM((E_PAD // NS // CHUNK, CHUNK), jnp.int32),
         pltpu.VMEM((E_PAD // NS // CHUNK, CHUNK), jnp.int32)]
        + [pltpu.VMEM((CHUNK, ZPADC), jnp.float32) for _ in range(4)]
        + [pltpu.SemaphoreType.DMA] * 8
    )

    tot = E_PAD // NS // CHUNK
    if cpt0 is None:
        cpt0 = tot // 2
    cpt1 = tot - cpt0
    assert cpt0 % 2 == 0 and cpt1 % 2 == 0

    def body(zpad, src2, dst2, zs_o, zd_o, srcv, dstv, *refs):
        bufs = refs[0:2]
        bufd = refs[2:4]
        gss, gsd = refs[4:6], refs[6:8]
        wss, wsd = refs[8:10], refs[10:12]
        cid = lax.axis_index("c")
        sid = lax.axis_index("s")
        cpt = jnp.where(cid == 0, cpt0, cpt1)
        cbase = jnp.where(cid == 0, sid * cpt0, NS * cpt0 + sid * cpt1)
        ebase = cbase * CHUNK
        pltpu.sync_copy(src2.at[pl.ds(cbase, cpt0)],
                        srcv.at[pl.ds(0, cpt0)])
        pltpu.sync_copy(dst2.at[pl.ds(cbase, cpt0)],
                        dstv.at[pl.ds(0, cpt0)])
        if cpt1 > cpt0:
            @pl.when(cid == 1)
            def _():
                pltpu.sync_copy(src2.at[pl.ds(cbase + cpt0, cpt1 - cpt0)],
                                srcv.at[pl.ds(cpt0, cpt1 - cpt0)])
                pltpu.sync_copy(dst2.at[pl.ds(cbase + cpt0, cpt1 - cpt0)],
                                dstv.at[pl.ds(cpt0, cpt1 - cpt0)])

        def g_start(k, b):
            pltpu.async_copy(zpad.at[srcv.at[k]], bufs[b], gss[b])
            pltpu.async_copy(zpad.at[dstv.at[k]], bufd[b], gsd[b])

        def g_wait(k, b):
            pltpu.make_async_copy(zpad.at[srcv.at[k]], bufs[b], gss[b]).wait()
            pltpu.make_async_copy(zpad.at[dstv.at[k]], bufd[b], gsd[b]).wait()

        def w_start(k, b):
            off = ebase + k * CHUNK
            pltpu.async_copy(bufs[b], zs_o.at[pl.ds(off, CHUNK)], wss[b])
            pltpu.async_copy(bufd[b], zd_o.at[pl.ds(off, CHUNK)], wsd[b])

        def w_wait(k, b):
            off = ebase + k * CHUNK
            pltpu.make_async_copy(bufs[b], zs_o.at[pl.ds(off, CHUNK)],
                                  wss[b]).wait()
            pltpu.make_async_copy(bufd[b], zd_o.at[pl.ds(off, CHUNK)],
                                  wsd[b]).wait()

        g_start(0, 0)
        g_start(1, 1)
        g_wait(0, 0)
        w_start(0, 0)

        @pl.loop(1, cpt // 2)
        def _(g):
            for b in range(2):
                k = g * 2 + b
                b1 = (b + 1) % 2
                w_wait(k - 2, b)
                g_start(k, b)
                g_wait(k - 1, b1)
                w_start(k - 1, b1)

        g_wait(cpt - 1, 1)
        w_start(cpt - 1, 1)
        w_wait(cpt - 2, 0)
        w_wait(cpt - 1, 1)

    return pl.kernel(body, out_type=out_type, mesh=mesh,
                     scratch_types=scratch,
                     compiler_params=pltpu.CompilerParams(
                         needs_layout_passes=False,
                         use_tc_tiling_on_sc=False))


def _tc1_body(aggp, degp, x, w1r, w1o, b1, w2r, h1_o, y2_o, rdeg_o):
    i = pl.program_id(0)
    agg = aggp[0] + aggp[1]
    degs = degp[:, pl.ds(i * BN, BN)]
    deg = lax.dot_general(degs, jnp.ones((NW, 1), jnp.float32),
                          (((0,), (0,)), ((), ())))
    rdeg = 1.0 / jnp.maximum(deg, 1.0)
    mean = agg * rdeg
    h1 = jnp.maximum(mean @ w1r[...] + b1[...][None, :] + x[...] @ w1o[...],
                     0.0)
    h1_o[...] = h1
    y2_o[...] = h1 @ w2r[...]
    rdeg_o[...] = rdeg


def _tc2_body(aggp, rdeg, h1, w2o, b2, wmu, wls, h2_o, y34_o):
    agg = aggp[0] + aggp[1]
    h2 = jnp.maximum(agg * rdeg[...] + b2[...][None, :] + h1[...] @ w2o[...],
                     0.0)
    h2_o[...] = h2
    y34_o[...] = jnp.concatenate(
        [h2 @ wmu[...], h2 @ wls[...], jnp.zeros((BN, 6), jnp.float32)],
        axis=1)


def _tc3_body(aggp, rdeg, h2, eps, wmu_o, bmu, wls_o, bls,
              wp1, bp1, wp2, bp2, wa1, ba1, wa2, ba2,
              pz_o, zpad_o, zsum_o, az_o):
    i = pl.program_id(0)
    m = (aggp[0] + aggp[1]) * rdeg[...]
    mu = m[:, :OC] + bmu[...][None, :] + h2[...] @ wmu_o[...]
    ls = jnp.minimum(m[:, OC:2 * OC] + bls[...][None, :] + h2[...] @ wls_o[...],
                     MAX_LOGSTD)
    z = mu + eps[...] * jnp.exp(ls)
    zpad_o[...] = jnp.concatenate(
        [z, jnp.zeros((BN, ZPADC - OC), jnp.float32)], axis=1)
    p = z @ wp1[...] + bp1[...][None, :]
    p = p @ wp2[...] + bp2[...][None, :]
    pm = jnp.max(p, axis=1, keepdims=True)
    pz_o[...] = p - pm - jnp.log(jnp.sum(jnp.exp(p - pm), axis=1,
                                         keepdims=True))

    @pl.when(i == 0)
    def _():
        zsum_o[...] = jnp.zeros_like(zsum_o)

    rows = lax.broadcasted_iota(jnp.int32, (BN, 1), 0)
    valid = rows < (N - i * BN)
    zsum_o[...] += jnp.sum(jnp.where(valid, z, 0.0), axis=0, keepdims=True)

    @pl.when(i == pl.num_programs(0) - 1)
    def _():
        pooled = zsum_o[...] * (1.0 / N)
        a = jnp.maximum(pooled @ wa1[...] + ba1[...][None, :], 0.0)
        a = a @ wa2[...] + ba2[...][None, :]
        am = jnp.max(a, axis=1, keepdims=True)
        az_o[...] = a - am - jnp.log(jnp.sum(jnp.exp(a - am), axis=1,
                                             keepdims=True))


def _tc4_body(zs, zd, pos_o):
    pos_o[...] = jax.nn.sigmoid(jnp.sum(zs[...] * zd[...], axis=1,
                                        keepdims=True))


def _full(shape):
    return pl.BlockSpec(shape, lambda i: tuple(0 for _ in shape))


_GRID_N = ACC_ROWS // BN


def _tc1(aggp, degp, x, w1r, w1o, b1, w2r):
    return pl.pallas_call(
        _tc1_body,
        grid=(_GRID_N,),
        in_specs=[
            pl.BlockSpec((NC, BN, D_IN), lambda i: (0, i, 0)),
            _full((NW, ACC_ROWS)),
            pl.BlockSpec((BN, D_IN), lambda i: (i, 0)),
            _full((D_IN, H1)), _full((D_IN, H1)), _full((H1,)),
            _full((H1, H2)),
        ],
        out_specs=[
            pl.BlockSpec((BN, H1), lambda i: (i, 0)),
            pl.BlockSpec((BN, H2), lambda i: (i, 0)),
            pl.BlockSpec((BN, 1), lambda i: (i, 0)),
        ],
        out_shape=[
            jax.ShapeDtypeStruct((N, H1), jnp.float32),
            jax.ShapeDtypeStruct((N, H2), jnp.float32),
            jax.ShapeDtypeStruct((N, 1), jnp.float32),
        ],
    )(aggp, degp, x, w1r, w1o, b1, w2r)


def _tc2(aggp, rdeg, h1, w2o, b2, wmu, wls):
    return pl.pallas_call(
        _tc2_body,
        grid=(_GRID_N,),
        in_specs=[
            pl.BlockSpec((NC, BN, H2), lambda i: (0, i, 0)),
            pl.BlockSpec((BN, 1), lambda i: (i, 0)),
            pl.BlockSpec((BN, H1), lambda i: (i, 0)),
            _full((H1, H2)), _full((H2,)),
            _full((H2, OC)), _full((H2, OC)),
        ],
        out_specs=[
            pl.BlockSpec((BN, H2), lambda i: (i, 0)),
            pl.BlockSpec((BN, 48), lambda i: (i, 0)),
        ],
        out_shape=[
            jax.ShapeDtypeStruct((N, H2), jnp.float32),
            jax.ShapeDtypeStruct((N, 48), jnp.float32),
        ],
    )(aggp, rdeg, h1, w2o, b2, wmu, wls)


def _tc3(aggp, rdeg, h2, eps, wmu_o, bmu, wls_o, bls,
         wp1, bp1, wp2, bp2, wa1, ba1, wa2, ba2):
    return pl.pallas_call(
        _tc3_body,
        grid=(_GRID_N,),
        in_specs=[
            pl.BlockSpec((NC, BN, 48), lambda i: (0, i, 0)),
            pl.BlockSpec((BN, 1), lambda i: (i, 0)),
            pl.BlockSpec((BN, H2), lambda i: (i, 0)),
            pl.BlockSpec((BN, OC), lambda i: (i, 0)),
            _full((H2, OC)), _full((OC,)), _full((H2, OC)), _full((OC,)),
            _full((OC, 5 * OC)), _full((5 * OC,)),
            _full((5 * OC, OC)), _full((OC,)),
            _full((OC, 5 * OC)), _full((5 * OC,)),
            _full((5 * OC, N_ACT)), _full((N_ACT,)),
        ],
        out_specs=[
            pl.BlockSpec((BN, OC), lambda i: (i, 0)),
            pl.BlockSpec((BN, ZPADC), lambda i: (i, 0)),
            pl.BlockSpec((1, OC), lambda i: (0, 0)),
            pl.BlockSpec((1, N_ACT), lambda i: (0, 0)),
        ],
        out_shape=[
            jax.ShapeDtypeStruct((N, OC), jnp.float32),
            jax.ShapeDtypeStruct((N, ZPADC), jnp.float32),
            jax.ShapeDtypeStruct((1, OC), jnp.float32),
            jax.ShapeDtypeStruct((1, N_ACT), jnp.float32),
        ],
    )(aggp, rdeg, h2, eps, wmu_o, bmu, wls_o, bls,
      wp1, bp1, wp2, bp2, wa1, ba1, wa2, ba2)


_BE = 4096


def _tc4(zs, zd):
    return pl.pallas_call(
        _tc4_body,
        grid=(E_PAD // _BE,),
        in_specs=[
            pl.BlockSpec((_BE, ZPADC), lambda i: (i, 0)),
            pl.BlockSpec((_BE, ZPADC), lambda i: (i, 0)),
        ],
        out_specs=pl.BlockSpec((_BE, 1), lambda i: (i, 0)),
        out_shape=jax.ShapeDtypeStruct((E_PAD, 1), jnp.float32),
    )(zs, zd)


_seg128 = _make_segsum(D_IN, True, 64, 4, 8, cpt0=312)
_seg168 = _make_segsum(H2, False, 64, 2, 4, cpt0=316)
_seg48 = _make_segsum(48, False, 128, 4, 8, cpt0=152)
_edge_gather = _make_edge_gather(cpt0=158)


def kernel(x, eps, W1_rel, W1_root, b1, W2_rel, W2_root, b2,
           Wmu_rel, Wmu_root, bmu, Wls_rel, Wls_root, bls,
           Wp1, bp1, Wp2, bp2, Wa1, ba1, Wa2, ba2, edge_index, batch):
    src = edge_index[0]
    dst = edge_index[1]
    pad = E_PAD - E + 2 * CHUNK
    srcf = jnp.concatenate([src, jnp.zeros((pad,), jnp.int32)])
    pad_dst = N + jnp.arange(pad, dtype=jnp.int32) % (ACC_ROWS - N)
    dstf = jnp.concatenate([dst, pad_dst])
    src2 = srcf.reshape(-1, CHUNK)
    dst2 = dstf.reshape(-1, CHUNK)
    z128 = jnp.zeros((RPT, D_IN), jnp.float32)
    z168 = jnp.zeros((RPT, H2), jnp.float32)
    z48 = jnp.zeros((RPT, 48), jnp.float32)

    agg1, degf = _seg128(x, srcf, dstf, z128)
    degp = degf.reshape(NW, ACC_ROWS)
    h1, y2, rdeg = _tc1(agg1, degp, x, W1_rel, W1_root, b1, W2_rel)
    (agg2,) = _seg168(y2, srcf, dstf, z168)
    h2, y34 = _tc2(agg2, rdeg, h1, W2_root, b2, Wmu_rel, Wls_rel)
    (agg34,) = _seg48(y34, srcf, dstf, z48)
    p_z, zpad, _zsum, a_z = _tc3(agg34, rdeg, h2, eps, Wmu_root, bmu,
                                 Wls_root, bls, Wp1, bp1, Wp2, bp2,
                                 Wa1, ba1, Wa2, ba2)
    zs, zd = _edge_gather(zpad, src2, dst2)
    pos2 = _tc4(zs, zd)

    z = zpad[:, :OC]
    pos_pred = pos2[:E, 0]
    return (p_z, a_z, z, pos_pred)


# speedup vs baseline: 1.0769x; 1.0769x over previous
import functools

import jax
import jax.numpy as jnp
from jax import lax
from jax.experimental import pallas as pl
from jax.experimental.pallas import tpu as pltpu
from jax.experimental.pallas import tpu_sc as plsc

N = 10000
E = 320000
D_IN = 128
OC = 21
H1 = 16 * OC
H2 = 8 * OC
N_ACT = 8
MAX_LOGSTD = 10.0

NC, NS = 2, 16
NW = NC * NS
CHUNK = 128
CPT = 80
E_PAD = NW * CPT * CHUNK
ACC_ROWS = 10240
RPT = ACC_ROWS // NS
BN = 1280
ZPADC = 32


def _make_segsum(d, with_hist, chunk, rr, ir, cpt0=None):
    tot = E_PAD // NS // chunk
    U = max(rr, ir)
    if cpt0 is None:
        cpt0 = tot // 2
    cpt1 = tot - cpt0
    assert cpt0 % U == 0 and cpt1 % U == 0 and ir >= 4
    mesh = plsc.VectorSubcoreMesh(core_axis_name="c", subcore_axis_name="s")
    out_type = [jax.ShapeDtypeStruct((NC, ACC_ROWS, d), jnp.float32)]
    if with_hist:
        out_type.append(jax.ShapeDtypeStruct((NW * ACC_ROWS,), jnp.float32))
    scratch = (
        [pltpu.VMEM((chunk,), jnp.int32) for _ in range(ir)]
        + [pltpu.VMEM((chunk,), jnp.int32) for _ in range(ir)]
        + [pltpu.VMEM((chunk, d), jnp.float32) for _ in range(rr)]
        + [pltpu.VMEM_SHARED((ACC_ROWS, d), jnp.float32)]
        + [pltpu.SemaphoreType.DMA] * (2 * ir + 2 * rr)
    )
    if with_hist:
        scratch.append(pltpu.VMEM((ACC_ROWS,), jnp.float32))

    def body(feat, srcf, dstf, zrows, *rest):
        if with_hist:
            (agg_o, deg_o), refs = rest[:2], rest[2:]
        else:
            (agg_o,), refs = rest[:1], rest[1:]
        srci = refs[0:ir]
        dsti = refs[ir:2 * ir]
        rows = refs[2 * ir:2 * ir + rr]
        acc = refs[2 * ir + rr]
        sems = refs[2 * ir + rr + 1:]
        isems = sems[0:ir]
        jsems = sems[ir:2 * ir]
        gsem = sems[2 * ir:2 * ir + rr]
        ssem = sems[2 * ir + rr:2 * ir + 2 * rr]
        if with_hist:
            histv = refs[-1]
        cid = lax.axis_index("c")
        sid = lax.axis_index("s")
        wid = cid * NS + sid
        cpt = jnp.where(cid == 0, cpt0, cpt1)
        cbase = jnp.where(cid == 0, sid * cpt0, NS * cpt0 + sid * cpt1)

        pltpu.sync_copy(zrows, acc.at[pl.ds(sid * RPT, RPT)])
        if with_hist:
            @pl.loop(0, ACC_ROWS // 16)
            def _(i):
                histv[pl.ds(i * 16, 16)] = jnp.zeros((16,), jnp.float32)

        plsc.subcore_barrier()

        ebase = cbase * chunk

        def i_start(kt, b):
            off = ebase + kt * chunk
            pltpu.async_copy(srcf.at[pl.ds(off, chunk)], srci[b], isems[b])
            pltpu.async_copy(dstf.at[pl.ds(off, chunk)], dsti[b], jsems[b])

        def i_wait(kt, b):
            off = ebase + kt * chunk
            pltpu.make_async_copy(srcf.at[pl.ds(off, chunk)], srci[b],
                                  isems[b]).wait()
            pltpu.make_async_copy(dstf.at[pl.ds(off, chunk)], dsti[b],
                                  jsems[b]).wait()

        def g_start(b, rb):
            pltpu.async_copy(feat.at[srci[b]], rows[rb], gsem[rb])

        def g_wait(b, rb):
            pltpu.make_async_copy(feat.at[srci[b]], rows[rb], gsem[rb]).wait()

        def s_start(b, rb):
            pltpu.async_copy(rows[rb], acc.at[dsti[b]], ssem[rb], add=True)

        def s_wait(b, rb):
            pltpu.make_async_copy(rows[rb], acc.at[dsti[b]], ssem[rb]).wait()

        ones = jnp.ones((16,), jnp.float32)

        def hist_upd(b):
            if with_hist:
                for j in range(chunk // 16):
                    idx = dsti[b][pl.ds(j * 16, 16)]
                    plsc.addupdate_scatter(histv, [idx], ones)

        def emit_step(sk, kt):
            if sk >= rr:
                s_wait((sk - rr) % ir, sk % rr)
            i_start(kt + 2, (sk + 2) % ir)
            i_wait(kt, sk % ir)
            g_start(sk % ir, sk % rr)
            hist_upd(sk % ir)
            if sk >= 1:
                g_wait((sk - 1) % ir, (sk - 1) % rr)
                s_start((sk - 1) % ir, (sk - 1) % rr)

        i_start(0, 0)
        i_start(1, 1)
        for k in range(U):
            emit_step(k, k)

        @pl.loop(1, cpt // U)
        def _(g):
            for u in range(U):
                emit_step(U + u, g * U + u)

        g_wait(ir - 1, rr - 1)
        s_start(ir - 1, rr - 1)
        i_wait(cpt, 0)
        i_wait(cpt + 1, 1)
        for j in range(rr):
            s_wait((ir - rr + j) % ir, j)

        if with_hist:
            pltpu.sync_copy(histv,
                            deg_o.at[pl.ds(wid * ACC_ROWS, ACC_ROWS)])

        plsc.subcore_barrier()
        pltpu.sync_copy(acc.at[pl.ds(sid * RPT, RPT)],
                        agg_o.at[cid, pl.ds(sid * RPT, RPT)])

    return pl.kernel(body, out_type=out_type, mesh=mesh,
                     scratch_types=scratch,
                     compiler_params=pltpu.CompilerParams(
                         needs_layout_passes=False,
                         use_tc_tiling_on_sc=False))


def _make_edge_gather(cpt0=None):
    mesh = plsc.VectorSubcoreMesh(core_axis_name="c", subcore_axis_name="s")
    out_type = [jax.ShapeDtypeStruct((E_PAD, ZPADC), jnp.float32),
                jax.ShapeDtypeStruct((E_PAD, ZPADC), jnp.float32)]
    scratch = (
        [pltpu.VMEM((E_PAD // NS // CHUNK, CHUNK), jnp.int32),
         pltpu.VMEM((E_PAD // NS // CHUNK, CHUNK), jnp.int32)]
        + [pltpu.VMEM((CHUNK, ZPADC), jnp.float32) for _ in range(4)]
        + [pltpu.SemaphoreType.DMA] * 8
    )

    tot = E_PAD // NS // CHUNK
    if cpt0 is None:
        cpt0 = tot // 2
    cpt1 = tot - cpt0
    assert cpt0 % 2 == 0 and cpt1 % 2 == 0

    def body(zpad, src2, dst2, zs_o, zd_o, srcv, dstv, *refs):
        bufs = refs[0:2]
        bufd = refs[2:4]
        gss, gsd = refs[4:6], refs[6:8]
        wss, wsd = refs[8:10], refs[10:12]
        cid = lax.axis_index("c")
        sid = lax.axis_index("s")
        cpt = jnp.where(cid == 0, cpt0, cpt1)
        cbase = jnp.where(cid == 0, sid * cpt0, NS * cpt0 + sid * cpt1)
        ebase = cbase * CHUNK
        pltpu.sync_copy(src2.at[pl.ds(cbase, cpt0)],
                        srcv.at[pl.ds(0, cpt0)])
        pltpu.sync_copy(dst2.at[pl.ds(cbase, cpt0)],
                        dstv.at[pl.ds(0, cpt0)])
        if cpt1 > cpt0:
            @pl.when(cid == 1)
            def _():
                pltpu.sync_copy(src2.at[pl.ds(cbase + cpt0, cpt1 - cpt0)],
                                srcv.at[pl.ds(cpt0, cpt1 - cpt0)])
                pltpu.sync_copy(dst2.at[pl.ds(cbase + cpt0, cpt1 - cpt0)],
                                dstv.at[pl.ds(cpt0, cpt1 - cpt0)])

        def g_start(k, b):
            pltpu.async_copy(zpad.at[srcv.at[k]], bufs[b], gss[b])
            pltpu.async_copy(zpad.at[dstv.at[k]], bufd[b], gsd[b])

        def g_wait(k, b):
            pltpu.make_async_copy(zpad.at[srcv.at[k]], bufs[b], gss[b]).wait()
            pltpu.make_async_copy(zpad.at[dstv.at[k]], bufd[b], gsd[b]).wait()

        def w_start(k, b):
            off = ebase + k * CHUNK
            pltpu.async_copy(bufs[b], zs_o.at[pl.ds(off, CHUNK)], wss[b])
            pltpu.async_copy(bufd[b], zd_o.at[pl.ds(off, CHUNK)], wsd[b])

        def w_wait(k, b):
            off = ebase + k * CHUNK
            pltpu.make_async_copy(bufs[b], zs_o.at[pl.ds(off, CHUNK)],
                                  wss[b]).wait()
            pltpu.make_async_copy(bufd[b], zd_o.at[pl.ds(off, CHUNK)],
                                  wsd[b]).wait()

        g_start(0, 0)
        g_start(1, 1)
        g_wait(0, 0)
        w_start(0, 0)

        @pl.loop(1, cpt // 2)
        def _(g):
            for b in range(2):
                k = g * 2 + b
                b1 = (b + 1) % 2
                w_wait(k - 2, b)
                g_start(k, b)
                g_wait(k - 1, b1)
                w_start(k - 1, b1)

        g_wait(cpt - 1, 1)
        w_start(cpt - 1, 1)
        w_wait(cpt - 2, 0)
        w_wait(cpt - 1, 1)

    return pl.kernel(body, out_type=out_type, mesh=mesh,
                     scratch_types=scratch,
                     compiler_params=pltpu.CompilerParams(
                         needs_layout_passes=False,
                         use_tc_tiling_on_sc=False))


def _tc1_body(aggp, degp, x, w1r, w1o, b1, w2r, h1_o, y2_o, rdeg_o):
    i = pl.program_id(0)
    agg = aggp[0] + aggp[1]
    degs = degp[:, pl.ds(i * BN, BN)]
    deg = lax.dot_general(degs, jnp.ones((NW, 1), jnp.float32),
                          (((0,), (0,)), ((), ())))
    rdeg = 1.0 / jnp.maximum(deg, 1.0)
    mean = agg * rdeg
    h1 = jnp.maximum(mean @ w1r[...] + b1[...][None, :] + x[...] @ w1o[...],
                     0.0)
    h1_o[...] = h1
    y2_o[...] = h1 @ w2r[...]
    rdeg_o[...] = rdeg


def _tc2_body(aggp, rdeg, h1, w2o, b2, wmu, wls, h2_o, y34_o):
    agg = aggp[0] + aggp[1]
    h2 = jnp.maximum(agg * rdeg[...] + b2[...][None, :] + h1[...] @ w2o[...],
                     0.0)
    h2_o[...] = h2
    y34_o[...] = jnp.concatenate(
        [h2 @ wmu[...], h2 @ wls[...], jnp.zeros((BN, 6), jnp.float32)],
        axis=1)


def _tc3_body(aggp, rdeg, h2, eps, wmu_o, bmu, wls_o, bls,
              wp1, bp1, wp2, bp2, wa1, ba1, wa2, ba2,
              pz_o, zpad_o, zsum_o, az_o):
    i = pl.program_id(0)
    m = (aggp[0] + aggp[1]) * rdeg[...]
    mu = m[:, :OC] + bmu[...][None, :] + h2[...] @ wmu_o[...]
    ls = jnp.minimum(m[:, OC:2 * OC] + bls[...][None, :] + h2[...] @ wls_o[...],
                     MAX_LOGSTD)
    z = mu + eps[...] * jnp.exp(ls)
    zpad_o[...] = jnp.concatenate(
        [z, jnp.zeros((BN, ZPADC - OC), jnp.float32)], axis=1)
    p = z @ wp1[...] + bp1[...][None, :]
    p = p @ wp2[...] + bp2[...][None, :]
    pm = jnp.max(p, axis=1, keepdims=True)
    pz_o[...] = p - pm - jnp.log(jnp.sum(jnp.exp(p - pm), axis=1,
                                         keepdims=True))

    @pl.when(i == 0)
    def _():
        zsum_o[...] = jnp.zeros_like(zsum_o)

    rows = lax.broadcasted_iota(jnp.int32, (BN, 1), 0)
    valid = rows < (N - i * BN)
    zsum_o[...] += jnp.sum(jnp.where(valid, z, 0.0), axis=0, keepdims=True)

    @pl.when(i == pl.num_programs(0) - 1)
    def _():
        pooled = zsum_o[...] * (1.0 / N)
        a = jnp.maximum(pooled @ wa1[...] + ba1[...][None, :], 0.0)
        a = a @ wa2[...] + ba2[...][None, :]
        am = jnp.max(a, axis=1, keepdims=True)
        az_o[...] = a - am - jnp.log(jnp.sum(jnp.exp(a - am), axis=1,
                                             keepdims=True))


def _tc4_body(zs, zd, pos_o):
    pos_o[...] = jax.nn.sigmoid(jnp.sum(zs[...] * zd[...], axis=1,
                                        keepdims=True))


def _full(shape):
    return pl.BlockSpec(shape, lambda i: tuple(0 for _ in shape))


_GRID_N = ACC_ROWS // BN


def _tc1(aggp, degp, x, w1r, w1o, b1, w2r):
    return pl.pallas_call(
        _tc1_body,
        grid=(_GRID_N,),
        in_specs=[
            pl.BlockSpec((NC, BN, D_IN), lambda i: (0, i, 0)),
            _full((NW, ACC_ROWS)),
            pl.BlockSpec((BN, D_IN), lambda i: (i, 0)),
            _full((D_IN, H1)), _full((D_IN, H1)), _full((H1,)),
            _full((H1, H2)),
        ],
        out_specs=[
            pl.BlockSpec((BN, H1), lambda i: (i, 0)),
            pl.BlockSpec((BN, H2), lambda i: (i, 0)),
            pl.BlockSpec((BN, 1), lambda i: (i, 0)),
        ],
        out_shape=[
            jax.ShapeDtypeStruct((N, H1), jnp.float32),
            jax.ShapeDtypeStruct((N, H2), jnp.float32),
            jax.ShapeDtypeStruct((N, 1), jnp.float32),
        ],
    )(aggp, degp, x, w1r, w1o, b1, w2r)


def _tc2(aggp, rdeg, h1, w2o, b2, wmu, wls):
    return pl.pallas_call(
        _tc2_body,
        grid=(_GRID_N,),
        in_specs=[
            pl.BlockSpec((NC, BN, H2), lambda i: (0, i, 0)),
            pl.BlockSpec((BN, 1), lambda i: (i, 0)),
            pl.BlockSpec((BN, H1), lambda i: (i, 0)),
            _full((H1, H2)), _full((H2,)),
            _full((H2, OC)), _full((H2, OC)),
        ],
        out_specs=[
            pl.BlockSpec((BN, H2), lambda i: (i, 0)),
            pl.BlockSpec((BN, 48), lambda i: (i, 0)),
        ],
        out_shape=[
            jax.ShapeDtypeStruct((N, H2), jnp.float32),
            jax.ShapeDtypeStruct((N, 48), jnp.float32),
        ],
    )(aggp, rdeg, h1, w2o, b2, wmu, wls)


def _tc3(aggp, rdeg, h2, eps, wmu_o, bmu, wls_o, bls,
         wp1, bp1, wp2, bp2, wa1, ba1, wa2, ba2):
    return pl.pallas_call(
        _tc3_body,
        grid=(_GRID_N,),
        in_specs=[
            pl.BlockSpec((NC, BN, 48), lambda i: (0, i, 0)),
            pl.BlockSpec((BN, 1), lambda i: (i, 0)),
            pl.BlockSpec((BN, H2), lambda i: (i, 0)),
            pl.BlockSpec((BN, OC), lambda i: (i, 0)),
            _full((H2, OC)), _full((OC,)), _full((H2, OC)), _full((OC,)),
            _full((OC, 5 * OC)), _full((5 * OC,)),
            _full((5 * OC, OC)), _full((OC,)),
            _full((OC, 5 * OC)), _full((5 * OC,)),
            _full((5 * OC, N_ACT)), _full((N_ACT,)),
        ],
        out_specs=[
            pl.BlockSpec((BN, OC), lambda i: (i, 0)),
            pl.BlockSpec((BN, ZPADC), lambda i: (i, 0)),
            pl.BlockSpec((1, OC), lambda i: (0, 0)),
            pl.BlockSpec((1, N_ACT), lambda i: (0, 0)),
        ],
        out_shape=[
            jax.ShapeDtypeStruct((N, OC), jnp.float32),
            jax.ShapeDtypeStruct((N, ZPADC), jnp.float32),
            jax.ShapeDtypeStruct((1, OC), jnp.float32),
            jax.ShapeDtypeStruct((1, N_ACT), jnp.float32),
        ],
    )(aggp, rdeg, h2, eps, wmu_o, bmu, wls_o, bls,
      wp1, bp1, wp2, bp2, wa1, ba1, wa2, ba2)


_BE = 4096


def _tc4(zs, zd):
    return pl.pallas_call(
        _tc4_body,
        grid=(E_PAD // _BE,),
        in_specs=[
            pl.BlockSpec((_BE, ZPADC), lambda i: (i, 0)),
            pl.BlockSpec((_BE, ZPADC), lambda i: (i, 0)),
        ],
        out_specs=pl.BlockSpec((_BE, 1), lambda i: (i, 0)),
        out_shape=jax.ShapeDtypeStruct((E_PAD, 1), jnp.float32),
    )(zs, zd)


_seg128 = _make_segsum(D_IN, True, 64, 4, 8, cpt0=296)
_seg168 = _make_segsum(H2, False, 64, 2, 4, cpt0=280)
_seg48 = _make_segsum(48, False, 128, 4, 8, cpt0=136)
_edge_gather = _make_edge_gather(cpt0=128)


def kernel(x, eps, W1_rel, W1_root, b1, W2_rel, W2_root, b2,
           Wmu_rel, Wmu_root, bmu, Wls_rel, Wls_root, bls,
           Wp1, bp1, Wp2, bp2, Wa1, ba1, Wa2, ba2, edge_index, batch):
    src = edge_index[0]
    dst = edge_index[1]
    pad = E_PAD - E + 2 * CHUNK
    srcf = jnp.concatenate([src, jnp.zeros((pad,), jnp.int32)])
    pad_dst = N + jnp.arange(pad, dtype=jnp.int32) % (ACC_ROWS - N)
    dstf = jnp.concatenate([dst, pad_dst])
    src2 = srcf.reshape(-1, CHUNK)
    dst2 = dstf.reshape(-1, CHUNK)
    z128 = jnp.zeros((RPT, D_IN), jnp.float32)
    z168 = jnp.zeros((RPT, H2), jnp.float32)
    z48 = jnp.zeros((RPT, 48), jnp.float32)

    agg1, degf = _seg128(x, srcf, dstf, z128)
    degp = degf.reshape(NW, ACC_ROWS)
    h1, y2, rdeg = _tc1(agg1, degp, x, W1_rel, W1_root, b1, W2_rel)
    (agg2,) = _seg168(y2, srcf, dstf, z168)
    h2, y34 = _tc2(agg2, rdeg, h1, W2_root, b2, Wmu_rel, Wls_rel)
    (agg34,) = _seg48(y34, srcf, dstf, z48)
    p_z, zpad, _zsum, a_z = _tc3(agg34, rdeg, h2, eps, Wmu_root, bmu,
                                 Wls_root, bls, Wp1, bp1, Wp2, bp2,
                                 Wa1, ba1, Wa2, ba2)
    zs, zd = _edge_gather(zpad, src2, dst2)
    pos2 = _tc4(zs, zd)

    z = zpad[:, :OC]
    pos_pred = pos2[:E, 0]
    return (p_z, a_z, z, pos_pred)
